# baseline (speedup 1.0000x reference)
import functools

import jax
import jax.numpy as jnp
from jax import lax
from jax.experimental import pallas as pl
from jax.experimental.pallas import tpu as pltpu
from jax.experimental.pallas import tpu_sc as plsc

B = 16384
D = 32
L = 16
NC = 2
NS = 16
NW = NC * NS
BPW = B // NW
CHUNK = 128
NCHUNK = BPW // CHUNK


@functools.partial(
    pl.kernel,
    out_type=jax.ShapeDtypeStruct((B,), jnp.float32),
    mesh=plsc.VectorSubcoreMesh(core_axis_name="c", subcore_axis_name="s"),
    compiler_params=pltpu.CompilerParams(
        needs_layout_passes=False, use_tc_tiling_on_sc=False),
    scratch_types=[
        pltpu.VMEM((NCHUNK, CHUNK), jnp.int32),
        pltpu.VMEM((NCHUNK, CHUNK), jnp.int32),
        pltpu.VMEM((BPW, D), jnp.float32),
        pltpu.VMEM((BPW, D), jnp.float32),
        pltpu.VMEM((BPW,), jnp.float32),
        pltpu.SemaphoreType.DMA,
    ],
)
def _cf_sc(user_hbm, item_hbm, uemb_hbm, iemb_hbm, out_hbm,
           uidx, iidx, urows, irows, outv, sem):
    wid = lax.axis_index("s") * NC + lax.axis_index("c")
    pltpu.sync_copy(user_hbm.at[pl.ds(wid * NCHUNK, NCHUNK)], uidx)
    pltpu.sync_copy(item_hbm.at[pl.ds(wid * NCHUNK, NCHUNK)], iidx)
    copies = []
    for j in range(NCHUNK):
        copies.append(pltpu.async_copy(
            uemb_hbm.at[uidx.at[j]], urows.at[pl.ds(j * CHUNK, CHUNK)], sem))
        copies.append(pltpu.async_copy(
            iemb_hbm.at[iidx.at[j]], irows.at[pl.ds(j * CHUNK, CHUNK)], sem))
    for cp in copies:
        cp.wait()

    lanes = lax.iota(jnp.int32, L)

    def group(g, carry):
        row = g * L + lanes
        acc = jnp.zeros((L,), jnp.float32)
        for d in range(D):
            col = jnp.full((L,), d, jnp.int32)
            u = plsc.load_gather(urows, [row, col])
            v = plsc.load_gather(irows, [row, col])
            acc = acc + u * v
        outv[pl.ds(pl.multiple_of(g * L, L), L)] = acc
        return carry

    lax.fori_loop(0, BPW // L, group, 0)
    pltpu.sync_copy(outv, out_hbm.at[pl.ds(wid * BPW, BPW)])


def kernel(user, item, user_emb, item_emb):
    u2 = user.reshape(B // CHUNK, CHUNK)
    i2 = item.reshape(B // CHUNK, CHUNK)
    return _cf_sc(u2, i2, user_emb, item_emb)
